# baseline (speedup 1.0000x reference)
import functools

import jax
import jax.numpy as jnp
from jax import lax
from jax.experimental import pallas as pl
from jax.experimental.pallas import tpu as pltpu
from jax.experimental.pallas import tpu_sc as plsc

N = 4194304
NUM_WORKERS = 32
E = N // NUM_WORKERS
C = 8192
L = 16

_LOG1P_COEFS = (
    1.4716139178361232e-06,
    0.9998477086047314,
    -0.49737329284895443,
    0.31574753794892985,
    -0.19035463580193973,
    0.08269142070032812,
    -0.017414116885094866,
)


def _bce_masked(z, t, m):
    u = jnp.exp(-jnp.abs(z))
    p = jnp.float32(_LOG1P_COEFS[-1])
    for c in _LOG1P_COEFS[-2::-1]:
        p = p * u + jnp.float32(c)
    loss = jnp.maximum(z, 0.0) - z * t + p
    return loss * m


def _sc_body(z_hbm, t_hbm, m_hbm, out_hbm, z_v, t_v, m_v, o_v):
    wid = lax.axis_index("s") * 2 + lax.axis_index("c")
    base = wid * E

    def chunk(k, _):
        off = base + k * C
        pltpu.sync_copy(z_hbm.at[pl.ds(off, C)], z_v)
        pltpu.sync_copy(t_hbm.at[pl.ds(off, C)], t_v)
        pltpu.sync_copy(m_hbm.at[pl.ds(off, C)], m_v)

        def grp(i, _):
            s = i * L
            z = z_v[pl.ds(s, L)]
            t = t_v[pl.ds(s, L)]
            m = m_v[pl.ds(s, L)]
            o_v[pl.ds(s, L)] = _bce_masked(z, t, m)
            return 0

        lax.fori_loop(0, C // L, grp, 0, unroll=4)
        pltpu.sync_copy(o_v, out_hbm.at[pl.ds(off, C)])
        return 0

    lax.fori_loop(0, E // C, chunk, 0)


@jax.jit
def _run(z, t, m):
    mesh = plsc.VectorSubcoreMesh(core_axis_name="c", subcore_axis_name="s")
    f = functools.partial(
        pl.kernel,
        mesh=mesh,
        out_type=jax.ShapeDtypeStruct((N,), jnp.float32),
        scratch_types=[
            pltpu.VMEM((C,), jnp.float32),
            pltpu.VMEM((C,), jnp.float32),
            pltpu.VMEM((C,), jnp.float32),
            pltpu.VMEM((C,), jnp.float32),
        ],
    )(_sc_body)
    return f(z, t, m)


def kernel(outputs, targets, precondition):
    return _run(outputs, targets, precondition.astype(jnp.float32))


# speedup vs baseline: 1.2601x; 1.2601x over previous
import functools

import jax
import jax.numpy as jnp
from jax import lax
from jax.experimental import pallas as pl
from jax.experimental.pallas import tpu as pltpu
from jax.experimental.pallas import tpu_sc as plsc

N = 4194304
NUM_WORKERS = 32
E = N // NUM_WORKERS
C = 16384
L = 16
SG = 32 * L

_SP_COEFS = (
    0.6934400245674142,
    -0.504522728113742,
    0.13579760108493755,
    -0.00839238496601271,
    -0.0035964622629846266,
    0.0008988160246652816,
    -8.253799605864947e-05,
    2.7965090629445915e-06,
)


def _sc_body(z_hbm, t_hbm, m_hbm, out_hbm, z_v, t_v, m_v, o_v):
    wid = lax.axis_index("s") * 2 + lax.axis_index("c")
    base = wid * E

    def chunk(k, _):
        off = pl.multiple_of(base + k * C, 256)
        moff = pl.multiple_of((base + k * C) // 32, 8)
        pltpu.sync_copy(z_hbm.at[pl.ds(off, C)], z_v)
        pltpu.sync_copy(t_hbm.at[pl.ds(off, C)], t_v)
        pltpu.sync_copy(m_hbm.at[pl.ds(moff, C // 32)], m_v)

        def supergroup(s, _):
            mbits = m_v[pl.ds(s * L, L)]
            for g in range(32):
                sl = pl.ds(s * SG + g * L, L)
                z = z_v[sl]
                t = t_v[sl]
                a = jnp.minimum(jnp.abs(z), 8.0)
                p = jnp.float32(_SP_COEFS[-1])
                for c in _SP_COEFS[-2::-1]:
                    p = p * a + jnp.float32(c)
                loss = jnp.maximum(z, 0.0) - z * t + p
                m = (lax.shift_right_logical(mbits, g) & 1).astype(jnp.float32)
                o_v[sl] = loss * m
            return 0

        lax.fori_loop(0, C // SG, supergroup, 0)
        pltpu.sync_copy(o_v, out_hbm.at[pl.ds(off, C)])
        return 0

    lax.fori_loop(0, E // C, chunk, 0)


@jax.jit
def _run(z, t, mpacked):
    mesh = plsc.VectorSubcoreMesh(core_axis_name="c", subcore_axis_name="s")
    f = functools.partial(
        pl.kernel,
        mesh=mesh,
        out_type=jax.ShapeDtypeStruct((N,), jnp.float32),
        scratch_types=[
            pltpu.VMEM((C,), jnp.float32),
            pltpu.VMEM((C,), jnp.float32),
            pltpu.VMEM((C // 32,), jnp.int32),
            pltpu.VMEM((C,), jnp.float32),
        ],
    )(_sc_body)
    return f(z, t, mpacked)


@jax.jit
def _pack_mask(precondition):
    m = precondition.reshape(-1, 32, L).astype(jnp.int32)
    shifted = m << jnp.arange(32, dtype=jnp.int32)[None, :, None]
    return shifted.sum(axis=1).reshape(-1)


def kernel(outputs, targets, precondition):
    return _run(outputs, targets, _pack_mask(precondition))
